# baseline (speedup 1.0000x reference)
import functools

import jax
import jax.numpy as jnp
from jax import lax
from jax.experimental import pallas as pl
from jax.experimental.pallas import tpu as pltpu
from jax.experimental.pallas import tpu_sc as plsc

N = 4096
P = 8
T = 4
D = 256
INNER = 512

_RAW_SIZES = [65, 64, 8, 129, 65, 129, 65, 65, 129, 257, 257, 17, 65]
_PAD_SIZES = [(s + 7) // 8 * 8 for s in _RAW_SIZES]
_OFFS = []
_acc = 0
for _s in _PAD_SIZES:
    _OFFS.append(_acc)
    _acc += _s
VTOT_RAW = _acc
VTOT = 1408

(OFF_SQ, OFF_KIND, OFF_PROMO, OFF_AP, OFF_AC, OFF_TP, OFF_TC, OFF_TAG,
 OFF_NS, OFF_LABEL, OFF_PATH, OFF_DEPTH, OFF_POS) = _OFFS

BN = 256
NH = N // 2
LANES = 16
A_CHUNK = 32
WORKERS = 32
RPW = NH // WORKERS
NCHUNK = RPW // A_CHUNK
W_BASE = 1.0
W_VAL = 1.0 / P


def _sc_build_c(row0, fs_hbm, ts_hbm, kind_hbm, promo_hbm, ap_hbm, ac_hbm,
                tp_hbm, tc_hbm, tag_hbm, ns_hbm, lab_hbm, path_hbm,
                dep_hbm, pos_hbm, c_hbm,
                bbuf, tagbuf, vbuf, cbuf0, cbuf1, sem_in, sem0, sem1):
    wid = lax.axis_index("s") * 2 + lax.axis_index("c")
    w0 = row0 + wid * RPW
    iota16 = lax.broadcasted_iota(jnp.int32, (LANES,), 0)

    base_refs = [fs_hbm, ts_hbm, kind_hbm, promo_hbm, ap_hbm, ac_hbm,
                 tp_hbm, tc_hbm]
    handles = []
    for i, ref in enumerate(base_refs):
        handles.append(pltpu.async_copy(
            ref.at[pl.ds(w0, RPW)], bbuf.at[pl.ds(i * RPW, RPW)], sem_in))
    handles.append(pltpu.async_copy(
        tag_hbm.at[pl.ds(w0 * T, RPW * T)], tagbuf, sem_in))
    val_refs = [ns_hbm, lab_hbm, path_hbm, dep_hbm, pos_hbm]
    for i, ref in enumerate(val_refs):
        handles.append(pltpu.async_copy(
            ref.at[pl.ds(w0 * P, RPW * P)],
            vbuf.at[pl.ds(i * RPW * P, RPW * P)], sem_in))

    def zero_row(a, carry):
        for j in range(VTOT // LANES):
            z = jnp.zeros((LANES,), jnp.float32)
            cbuf0[a, pl.ds(j * LANES, LANES)] = z
            cbuf1[a, pl.ds(j * LANES, LANES)] = z
        return carry

    lax.fori_loop(0, A_CHUNK, zero_row, 0)
    for h in handles:
        h.wait()

    base_offs = [OFF_SQ, OFF_SQ, OFF_KIND, OFF_PROMO, OFF_AP, OFF_AC,
                 OFF_TP, OFF_TC]
    val_offs = [OFF_NS, OFF_LABEL, OFF_PATH, OFF_DEPTH, OFF_POS]

    def scatter_chunk(ci, cbuf, sign):
        def tile(t, carry):
            a0 = ci * A_CHUNK + t * LANES
            rows = t * LANES + iota16
            acts = a0 + iota16
            wb = jnp.full((LANES,), sign * W_BASE, jnp.float32)
            wv = jnp.full((LANES,), sign * W_VAL, jnp.float32)
            for f in range(8):
                idx = bbuf[pl.ds(f * RPW + a0, LANES)] + base_offs[f]
                plsc.addupdate_scatter(cbuf, [rows, idx], wb)
            for t_i in range(T):
                idx = plsc.load_gather(tagbuf, [acts * T + t_i]) + OFF_TAG
                plsc.addupdate_scatter(cbuf, [rows, idx], wb)
            for v_i in range(5):
                for p_i in range(P):
                    idx = plsc.load_gather(
                        vbuf, [v_i * RPW * P + acts * P + p_i]) + val_offs[v_i]
                    plsc.addupdate_scatter(cbuf, [rows, idx], wv)
            return carry
        lax.fori_loop(0, A_CHUNK // LANES, tile, 0)

    cbufs = [cbuf0, cbuf1]
    sems = [sem0, sem1]
    o0 = wid * RPW
    out_handles = []
    for ci in range(NCHUNK):
        scatter_chunk(ci, cbufs[ci % 2], 1.0)
        out_handles.append(pltpu.async_copy(
            cbufs[ci % 2], c_hbm.at[pl.ds(o0 + ci * A_CHUNK, A_CHUNK)],
            sems[ci % 2]))
    for h in out_handles:
        h.wait()


def _tc_body(c_ref, af_ref, vf_ref, thi_ref, tlo_ref,
             w1f_ref, b1f_ref, w2f_ref, b2f_ref,
             w1v_ref, b1v_ref, w2v_ref, b2v_ref,
             gamma_ref, beta_ref, wo1_ref, bo1_ref, wo2_ref, bo2_ref,
             out_ref):
    cbf = c_ref[...]
    emb = (jnp.dot(cbf, thi_ref[...], preferred_element_type=jnp.float32)
           + jnp.dot(cbf, tlo_ref[...], preferred_element_type=jnp.float32))

    hf = jax.nn.gelu(jnp.dot(af_ref[...], w1f_ref[...],
                             preferred_element_type=jnp.float32) + b1f_ref[...])
    flag = jnp.dot(hf.astype(jnp.bfloat16), w2f_ref[...],
                   preferred_element_type=jnp.float32) + b2f_ref[...]

    x2 = vf_ref[...].reshape(P * BN, 10)
    h2 = jax.nn.gelu(jnp.dot(x2, w1v_ref[...],
                             preferred_element_type=jnp.float32) + b1v_ref[...])
    acc = jnp.sum(h2.reshape(P, BN, INNER), axis=0)
    val = jnp.dot((acc * jnp.float32(1.0 / P)).astype(jnp.bfloat16),
                  w2v_ref[...],
                  preferred_element_type=jnp.float32) + b2v_ref[...]

    h = emb + flag + val
    mu = jnp.mean(h, axis=-1, keepdims=True)
    dev = h - mu
    var = jnp.mean(dev * dev, axis=-1, keepdims=True)
    hn = dev * lax.rsqrt(var + jnp.float32(1e-5)) * gamma_ref[...] \
        + beta_ref[...]

    ho = jax.nn.gelu(jnp.dot(hn.astype(jnp.bfloat16), wo1_ref[...],
                             preferred_element_type=jnp.float32) + bo1_ref[...])
    out_ref[...] = jnp.dot(ho.astype(jnp.bfloat16), wo2_ref[...],
                           preferred_element_type=jnp.float32) + bo2_ref[...]


def _pad_rows(tbl, padded):
    v = tbl.shape[0]
    if padded == v:
        return tbl
    return jnp.concatenate(
        [tbl, jnp.zeros((padded - v, tbl.shape[1]), tbl.dtype)], axis=0)


def kernel(from_square_idx, to_square_idx, action_kind_idx, promotion_idx,
           actor_piece_idx, actor_class_idx, target_piece_idx,
           target_class_idx, tag_idx, value_namespace_idx, value_label_idx,
           value_path_idx, value_depth_idx, value_position_idx,
           action_features, value_features,
           square_table, kind_table, promo_table, actor_piece_table,
           actor_class_table, target_piece_table, target_class_table,
           tag_table, ns_table, label_table, path_table, depth_table,
           pos_table,
           W1f, b1f, W2f, b2f, W1v, b1v, W2v, b2v,
           gamma, beta, Wo1, bo1, Wo2, bo2):
    mesh = plsc.VectorSubcoreMesh(core_axis_name="c", subcore_axis_name="s")
    idx_args = (from_square_idx.astype(jnp.int32),
                to_square_idx.astype(jnp.int32),
                action_kind_idx.astype(jnp.int32),
                promotion_idx.astype(jnp.int32),
                actor_piece_idx.astype(jnp.int32),
                actor_class_idx.astype(jnp.int32),
                target_piece_idx.astype(jnp.int32),
                target_class_idx.astype(jnp.int32),
                tag_idx.astype(jnp.int32).reshape(-1),
                value_namespace_idx.astype(jnp.int32).reshape(-1),
                value_label_idx.astype(jnp.int32).reshape(-1),
                value_path_idx.astype(jnp.int32).reshape(-1),
                value_depth_idx.astype(jnp.int32).reshape(-1),
                value_position_idx.astype(jnp.int32).reshape(-1))
    c_halves = []
    for h in range(2):
        sc_call = functools.partial(
            pl.kernel, mesh=mesh,
            out_type=jax.ShapeDtypeStruct((NH, VTOT), jnp.float32),
            scratch_types=[
                pltpu.VMEM((8 * RPW,), jnp.int32),
                pltpu.VMEM((RPW * T,), jnp.int32),
                pltpu.VMEM((5 * RPW * P,), jnp.int32),
                pltpu.VMEM((A_CHUNK, VTOT), jnp.float32),
                pltpu.VMEM((A_CHUNK, VTOT), jnp.float32),
                pltpu.SemaphoreType.DMA,
                pltpu.SemaphoreType.DMA,
                pltpu.SemaphoreType.DMA,
            ],
            compiler_params=pltpu.CompilerParams(needs_layout_passes=False),
        )(functools.partial(_sc_build_c, h * NH))
        c_halves.append(sc_call(*idx_args))

    tables = [square_table, kind_table, promo_table, actor_piece_table,
              actor_class_table, target_piece_table, target_class_table,
              tag_table, ns_table, label_table, path_table, depth_table,
              pos_table]
    tcat = jnp.concatenate(
        [_pad_rows(t, s) for t, s in zip(tables, _PAD_SIZES)]
        + [jnp.zeros((VTOT - VTOT_RAW, D), jnp.float32)], axis=0)
    thi = tcat.astype(jnp.bfloat16)
    tlo = (tcat - thi.astype(jnp.float32)).astype(jnp.bfloat16)
    vf_t = jnp.transpose(value_features, (1, 0, 2))
    W2f = W2f.astype(jnp.bfloat16)
    W2v = W2v.astype(jnp.bfloat16)
    Wo1 = Wo1.astype(jnp.bfloat16)
    Wo2 = Wo2.astype(jnp.bfloat16)

    grid = (NH // BN,)
    full = lambda a: pl.BlockSpec(a.shape, lambda i: (0,) * a.ndim)
    outs = []
    for h in range(2):
        hb = h * (NH // BN)
        c_bf = c_halves[h].astype(jnp.bfloat16)
        outs.append(pl.pallas_call(
            _tc_body,
            grid=grid,
            in_specs=[
                pl.BlockSpec((BN, VTOT), lambda i: (i, 0)),
                pl.BlockSpec((BN, 8),
                             lambda i, hb=hb: (i + hb, 0)),
                pl.BlockSpec((P, BN, 10),
                             lambda i, hb=hb: (0, i + hb, 0)),
                full(thi), full(tlo), full(W1f), full(b1f), full(W2f),
                full(b2f), full(W1v), full(b1v), full(W2v), full(b2v),
                full(gamma), full(beta), full(Wo1), full(bo1), full(Wo2),
                full(bo2),
            ],
            out_specs=pl.BlockSpec((BN, D), lambda i: (i, 0)),
            out_shape=jax.ShapeDtypeStruct((NH, D), jnp.float32),
            compiler_params=pltpu.CompilerParams(
                dimension_semantics=("parallel",)),
        )(c_bf, action_features, vf_t, thi, tlo, W1f, b1f, W2f, b2f,
          W1v, b1v, W2v, b2v, gamma, beta, Wo1, bo1, Wo2, bo2))
    return jnp.concatenate(outs, axis=0)


# speedup vs baseline: 1.1153x; 1.1153x over previous
import functools

import jax
import jax.numpy as jnp
from jax import lax
from jax.experimental import pallas as pl
from jax.experimental.pallas import tpu as pltpu
from jax.experimental.pallas import tpu_sc as plsc

N = 4096
P = 8
T = 4
D = 256
INNER = 512

_RAW_SIZES = [65, 64, 8, 129, 65, 129, 65, 65, 129, 257, 257, 17, 65]
_PAD_SIZES = [(s + 7) // 8 * 8 for s in _RAW_SIZES]
_OFFS = []
_acc = 0
for _s in _PAD_SIZES:
    _OFFS.append(_acc)
    _acc += _s
VTOT_RAW = _acc
VTOT = 1408

(OFF_SQ, OFF_KIND, OFF_PROMO, OFF_AP, OFF_AC, OFF_TP, OFF_TC, OFF_TAG,
 OFF_NS, OFF_LABEL, OFF_PATH, OFF_DEPTH, OFF_POS) = _OFFS

BN = 256
LANES = 16
A_CHUNK = 32
WORKERS = 32
RPW = N // WORKERS
NCHUNK = RPW // A_CHUNK
W_BASE = 1.0
W_VAL = 1.0 / P

FLAT_TAG = 8 * N
FLAT_VAL = FLAT_TAG + N * T


def _sc_build_c(flat_hbm, c_hbm,
                bbuf, tagbuf, vbuf, cbuf0, cbuf1, sem_in, sem0, sem1):
    wid = lax.axis_index("s") * 2 + lax.axis_index("c")
    w0 = wid * RPW
    iota16 = lax.broadcasted_iota(jnp.int32, (LANES,), 0)

    handles = []
    for i in range(8):
        handles.append(pltpu.async_copy(
            flat_hbm.at[pl.ds(i * N + w0, RPW)],
            bbuf.at[pl.ds(i * RPW, RPW)], sem_in))
    handles.append(pltpu.async_copy(
        flat_hbm.at[pl.ds(FLAT_TAG + w0 * T, RPW * T)], tagbuf, sem_in))
    for i in range(5):
        handles.append(pltpu.async_copy(
            flat_hbm.at[pl.ds(FLAT_VAL + i * N * P + w0 * P, RPW * P)],
            vbuf.at[pl.ds(i * RPW * P, RPW * P)], sem_in))

    def zero_row(a, carry):
        for j in range(VTOT // LANES):
            z = jnp.zeros((LANES,), jnp.float32)
            cbuf0[a, pl.ds(j * LANES, LANES)] = z
            cbuf1[a, pl.ds(j * LANES, LANES)] = z
        return carry

    lax.fori_loop(0, A_CHUNK, zero_row, 0)
    for h in handles:
        h.wait()

    base_offs = [OFF_SQ, OFF_SQ, OFF_KIND, OFF_PROMO, OFF_AP, OFF_AC,
                 OFF_TP, OFF_TC]
    val_offs = [OFF_NS, OFF_LABEL, OFF_PATH, OFF_DEPTH, OFF_POS]

    def scatter_chunk(ci, cbuf, sign):
        def tile(t, carry):
            a0 = ci * A_CHUNK + t * LANES
            rows = t * LANES + iota16
            acts = a0 + iota16
            wb = jnp.full((LANES,), sign * W_BASE, jnp.float32)
            wv = jnp.full((LANES,), sign * W_VAL, jnp.float32)
            for f in range(8):
                idx = bbuf[pl.ds(f * RPW + a0, LANES)] + base_offs[f]
                plsc.addupdate_scatter(cbuf, [rows, idx], wb)
            for t_i in range(T):
                idx = plsc.load_gather(tagbuf, [acts * T + t_i]) + OFF_TAG
                plsc.addupdate_scatter(cbuf, [rows, idx], wb)
            for v_i in range(5):
                for p_i in range(P):
                    idx = plsc.load_gather(
                        vbuf, [v_i * RPW * P + acts * P + p_i]) + val_offs[v_i]
                    plsc.addupdate_scatter(cbuf, [rows, idx], wv)
            return carry
        lax.fori_loop(0, A_CHUNK // LANES, tile, 0)

    cbufs = [cbuf0, cbuf1]
    sems = [sem0, sem1]
    out_handles = [None, None]
    for ci in range(NCHUNK):
        b = ci % 2
        if out_handles[b] is not None:
            out_handles[b].wait()
            scatter_chunk(ci - 2, cbufs[b], -1.0)
        scatter_chunk(ci, cbufs[b], 1.0)
        out_handles[b] = pltpu.async_copy(
            cbufs[b], c_hbm.at[pl.ds(w0 + ci * A_CHUNK, A_CHUNK)], sems[b])
    for b in range(2):
        if out_handles[b] is not None:
            out_handles[b].wait()


def _tc_body(c_ref, af_ref, vf_ref, thi_ref, tlo_ref,
             w1f_ref, b1f_ref, w2f_ref, b2f_ref,
             w1v_ref, b1v_ref, w2v_ref, b2v_ref,
             gamma_ref, beta_ref, wo1_ref, bo1_ref, wo2_ref, bo2_ref,
             out_ref):
    cbf = c_ref[...]
    emb = (jnp.dot(cbf, thi_ref[...], preferred_element_type=jnp.float32)
           + jnp.dot(cbf, tlo_ref[...], preferred_element_type=jnp.float32))

    hf = jax.nn.gelu(jnp.dot(af_ref[...], w1f_ref[...],
                             preferred_element_type=jnp.float32) + b1f_ref[...])
    flag = jnp.dot(hf.astype(jnp.bfloat16), w2f_ref[...],
                   preferred_element_type=jnp.float32) + b2f_ref[...]

    x2 = vf_ref[...].reshape(P * BN, 10)
    h2 = jax.nn.gelu(jnp.dot(x2, w1v_ref[...],
                             preferred_element_type=jnp.float32) + b1v_ref[...])
    acc = jnp.sum(h2.reshape(P, BN, INNER), axis=0)
    val = jnp.dot((acc * jnp.float32(1.0 / P)).astype(jnp.bfloat16),
                  w2v_ref[...],
                  preferred_element_type=jnp.float32) + b2v_ref[...]

    h = emb + flag + val
    mu = jnp.mean(h, axis=-1, keepdims=True)
    dev = h - mu
    var = jnp.mean(dev * dev, axis=-1, keepdims=True)
    hn = dev * lax.rsqrt(var + jnp.float32(1e-5)) * gamma_ref[...] \
        + beta_ref[...]

    ho = jax.nn.gelu(jnp.dot(hn.astype(jnp.bfloat16), wo1_ref[...],
                             preferred_element_type=jnp.float32) + bo1_ref[...])
    out_ref[...] = jnp.dot(ho.astype(jnp.bfloat16), wo2_ref[...],
                           preferred_element_type=jnp.float32) + bo2_ref[...]


def _pad_rows(tbl, padded):
    v = tbl.shape[0]
    if padded == v:
        return tbl
    return jnp.concatenate(
        [tbl, jnp.zeros((padded - v, tbl.shape[1]), tbl.dtype)], axis=0)


def kernel(from_square_idx, to_square_idx, action_kind_idx, promotion_idx,
           actor_piece_idx, actor_class_idx, target_piece_idx,
           target_class_idx, tag_idx, value_namespace_idx, value_label_idx,
           value_path_idx, value_depth_idx, value_position_idx,
           action_features, value_features,
           square_table, kind_table, promo_table, actor_piece_table,
           actor_class_table, target_piece_table, target_class_table,
           tag_table, ns_table, label_table, path_table, depth_table,
           pos_table,
           W1f, b1f, W2f, b2f, W1v, b1v, W2v, b2v,
           gamma, beta, Wo1, bo1, Wo2, bo2):
    mesh = plsc.VectorSubcoreMesh(core_axis_name="c", subcore_axis_name="s")
    flat_idx = jnp.concatenate(
        [from_square_idx.astype(jnp.int32),
         to_square_idx.astype(jnp.int32),
         action_kind_idx.astype(jnp.int32),
         promotion_idx.astype(jnp.int32),
         actor_piece_idx.astype(jnp.int32),
         actor_class_idx.astype(jnp.int32),
         target_piece_idx.astype(jnp.int32),
         target_class_idx.astype(jnp.int32),
         tag_idx.astype(jnp.int32).reshape(-1),
         value_namespace_idx.astype(jnp.int32).reshape(-1),
         value_label_idx.astype(jnp.int32).reshape(-1),
         value_path_idx.astype(jnp.int32).reshape(-1),
         value_depth_idx.astype(jnp.int32).reshape(-1),
         value_position_idx.astype(jnp.int32).reshape(-1)])
    sc_call = functools.partial(
        pl.kernel, mesh=mesh,
        out_type=jax.ShapeDtypeStruct((N, VTOT), jnp.float32),
        scratch_types=[
            pltpu.VMEM((8 * RPW,), jnp.int32),
            pltpu.VMEM((RPW * T,), jnp.int32),
            pltpu.VMEM((5 * RPW * P,), jnp.int32),
            pltpu.VMEM((A_CHUNK, VTOT), jnp.float32),
            pltpu.VMEM((A_CHUNK, VTOT), jnp.float32),
            pltpu.SemaphoreType.DMA,
            pltpu.SemaphoreType.DMA,
            pltpu.SemaphoreType.DMA,
        ],
        compiler_params=pltpu.CompilerParams(needs_layout_passes=False),
    )(_sc_build_c)
    c = sc_call(flat_idx)

    tables = [square_table, kind_table, promo_table, actor_piece_table,
              actor_class_table, target_piece_table, target_class_table,
              tag_table, ns_table, label_table, path_table, depth_table,
              pos_table]
    tcat = jnp.concatenate(
        [_pad_rows(t, s) for t, s in zip(tables, _PAD_SIZES)]
        + [jnp.zeros((VTOT - VTOT_RAW, D), jnp.float32)], axis=0)
    thi = tcat.astype(jnp.bfloat16)
    tlo = (tcat - thi.astype(jnp.float32)).astype(jnp.bfloat16)
    vf_t = jnp.transpose(value_features, (1, 0, 2))
    W2f = W2f.astype(jnp.bfloat16)
    W2v = W2v.astype(jnp.bfloat16)
    Wo1 = Wo1.astype(jnp.bfloat16)
    Wo2 = Wo2.astype(jnp.bfloat16)

    grid = (N // BN,)
    full = lambda a: pl.BlockSpec(a.shape, lambda i: (0,) * a.ndim)
    c_bf = c.astype(jnp.bfloat16)
    out = pl.pallas_call(
        _tc_body,
        grid=grid,
        in_specs=[
            pl.BlockSpec((BN, VTOT), lambda i: (i, 0)),
            pl.BlockSpec((BN, 8), lambda i: (i, 0)),
            pl.BlockSpec((P, BN, 10), lambda i: (0, i, 0)),
            full(thi), full(tlo), full(W1f), full(b1f), full(W2f), full(b2f),
            full(W1v), full(b1v), full(W2v), full(b2v),
            full(gamma), full(beta), full(Wo1), full(bo1), full(Wo2),
            full(bo2),
        ],
        out_specs=pl.BlockSpec((BN, D), lambda i: (i, 0)),
        out_shape=jax.ShapeDtypeStruct((N, D), jnp.float32),
        compiler_params=pltpu.CompilerParams(
            dimension_semantics=("parallel",)),
    )(c_bf, action_features, vf_t, thi, tlo, W1f, b1f, W2f, b2f,
      W1v, b1v, W2v, b2v, gamma, beta, Wo1, bo1, Wo2, bo2)
    return out
